# baseline (speedup 1.0000x reference)
import jax
import jax.numpy as jnp
from jax.experimental import pallas as pl
from jax.experimental.pallas import tpu as pltpu

N = 10000
NUM_SYMPS = 360
RB = 256
NB = 40
TAIL = N - (NB - 1) * RB
NSLOT = 3
LOOK = 2
NUM_HERBS = 753
NHID = 64
DIM = 64


def _dot(a, b, dn=None):
    if dn is None:
        dn = (((1,), (0,)), ((), ()))
    return jax.lax.dot_general(a, b, dimension_numbers=dn,
                               precision=jax.lax.Precision.DEFAULT,
                               preferred_element_type=jnp.float32)


_DN_T = (((1,), (1,)), ((), ()))


def _s1_kernel(x_ref, w1_ref, s1_ref):
    s1_ref[...] = _dot(x_ref[...], w1_ref[...])


def _mega_kernel(s1_ref, b1_ref, w2_ref, b2_ref, wsh_ref, bsh_ref,
                 whc_ref, bhc_ref, adj_ref, sh_ref, hct_ref,
                 s2_ref, ring_ref, abuf_ref, sems):
    i = pl.program_id(0)

    def _copy(q, full_blk):
        b = jax.lax.rem(q, NB)
        slot = jax.lax.rem(q, NSLOT)
        if full_blk:
            return pltpu.make_async_copy(
                adj_ref.at[pl.ds(b * RB, RB), :],
                abuf_ref.at[slot], sems.at[slot])
        return pltpu.make_async_copy(
            adj_ref.at[pl.ds((NB - 1) * RB, TAIL), :],
            abuf_ref.at[slot, pl.ds(0, TAIL), :], sems.at[slot])

    def issue(q):
        b = jax.lax.rem(q, NB)

        @pl.when(b < NB - 1)
        def _():
            _copy(q, True).start()

        @pl.when(b == NB - 1)
        def _():
            _copy(q, False).start()

    def wait(q):
        b = jax.lax.rem(q, NB)

        @pl.when(b < NB - 1)
        def _():
            _copy(q, True).wait()

        @pl.when(b == NB - 1)
        def _():
            _copy(q, False).wait()

    @pl.when(i == 0)
    def _():
        for q in range(LOOK):
            issue(q)

    @pl.when(i + LOOK < 2 * NB)
    def _():
        issue(i + LOOK)

    wait(i)
    slot = jax.lax.rem(i, NSLOT)
    adj_blk = abuf_ref[slot]

    @pl.when(i < NB)
    def _():
        h = jnp.maximum(_dot(adj_blk, s1_ref[...]) + b1_ref[...], 0.0)
        s2_ref[pl.ds(jax.lax.rem(i, NB) * RB, RB), :] = _dot(h, w2_ref[...])

    @pl.when(i >= NB)
    def _():
        k = i - NB
        h2 = _dot(adj_blk, s2_ref[:N, :]) + b2_ref[...]

        @pl.when(k == 0)
        def _():
            logits_s = _dot(h2, wsh_ref[...], _DN_T) + bsh_ref[...]
            sh_ref[:RB, :] = jax.nn.sigmoid(logits_s)

        @pl.when(k == 1)
        def _():
            logits_s = (_dot(h2[:NUM_SYMPS - RB], wsh_ref[...], _DN_T)
                        + bsh_ref[...])
            sh_ref[RB:, :] = jax.nn.sigmoid(logits_s)

        logits_t = _dot(whc_ref[...], h2, _DN_T) + bhc_ref[...]
        cur = jax.nn.sigmoid(logits_t)
        off = NUM_SYMPS - RB
        w = 2 * RB - NUM_SYMPS

        @pl.when(k > 1)
        def _():
            prev = ring_ref[(k - 1) % 2]
            hct_ref[:, :w] = prev[:, off:]
            hct_ref[:, w:] = cur[:, :off]

        ring_ref[k % 2] = cur


@jax.jit
def kernel(x, adj, W1, b1, W2, b2, Wsh, bsh, Whc, bhc):
    s1 = pl.pallas_call(
        _s1_kernel,
        out_shape=jax.ShapeDtypeStruct((N, NHID), jnp.float32),
    )(x, W1)

    full = lambda shape: pl.BlockSpec(shape, lambda i: (0, 0))

    sh, hct = pl.pallas_call(
        _mega_kernel,
        grid=(2 * NB,),
        in_specs=[
            full((N, NHID)),
            full((1, NHID)),
            full((NHID, DIM)),
            full((1, DIM)),
            full((NUM_HERBS, DIM)),
            full((1, NUM_HERBS)),
            full((NUM_HERBS, DIM)),
            full((NUM_HERBS, 1)),
            pl.BlockSpec(memory_space=pltpu.HBM),
        ],
        out_specs=[
            pl.BlockSpec((NUM_SYMPS, NUM_HERBS), lambda i: (0, 0)),
            pl.BlockSpec(
                (NUM_HERBS, RB),
                lambda i: (0, jnp.clip(i - NB - 2, 0,
                                       (N - NUM_SYMPS - 1) // RB))),
        ],
        out_shape=[
            jax.ShapeDtypeStruct((NUM_SYMPS, NUM_HERBS), jnp.float32),
            jax.ShapeDtypeStruct((NUM_HERBS, N - NUM_SYMPS), jnp.float32),
        ],
        scratch_shapes=[
            pltpu.VMEM((NB * RB, DIM), jnp.float32),
            pltpu.VMEM((2, NUM_HERBS, RB), jnp.float32),
            pltpu.VMEM((NSLOT, RB, N), jnp.float32),
            pltpu.SemaphoreType.DMA((NSLOT,)),
        ],
    )(s1, b1.reshape(1, NHID), W2, b2.reshape(1, DIM),
      Wsh, bsh.reshape(1, NUM_HERBS), Whc, bhc.reshape(NUM_HERBS, 1), adj)

    return (sh, hct.T)
